# baseline (speedup 1.0000x reference)
import functools

import jax
import jax.numpy as jnp
from jax import lax
from jax.experimental import pallas as pl
from jax.experimental.pallas import tpu as pltpu
from jax.experimental.pallas import tpu_sc as plsc

N_TOKENS = 8192
N_EMBD = 4096
NUM_EXPERTS = 64
TOP_K = 8

BM = 512
BK = 2048
KB = N_EMBD // BK

_NC = 2
_NS = 16
_L = 16
_NW = _NC * _NS
_ROWS_PER_W = N_TOKENS // _NW
_GROUPS = _ROWS_PER_W // _L
_ZN = _ROWS_PER_W * NUM_EXPERTS
_IN = _ROWS_PER_W * TOP_K

_INTERPRET = False


def _matmul_body(x_ref, w_ref, b_ref, noise_ref, z_ref, acc_ref):
    k = pl.program_id(1)

    @pl.when(k == 0)
    def _():
        acc_ref[...] = jnp.zeros_like(acc_ref)

    acc_ref[...] += jnp.dot(x_ref[...], w_ref[...],
                            preferred_element_type=jnp.float32)

    @pl.when(k == KB - 1)
    def _():
        acc = acc_ref[...] + b_ref[...]
        logits = acc[:, :NUM_EXPERTS]
        nlog = acc[:, NUM_EXPERTS:]
        softplus = jnp.maximum(nlog, 0.0) + jnp.log1p(jnp.exp(-jnp.abs(nlog)))
        z_ref[...] = logits + noise_ref[...] * softplus


def _noisy_logits(x, w, b, noise):
    grid = (N_TOKENS // BM, KB)
    return pl.pallas_call(
        _matmul_body,
        grid=grid,
        in_specs=[
            pl.BlockSpec((BM, BK), lambda i, j: (i, j)),
            pl.BlockSpec((BK, 2 * NUM_EXPERTS), lambda i, j: (j, 0)),
            pl.BlockSpec((1, 2 * NUM_EXPERTS), lambda i, j: (0, 0)),
            pl.BlockSpec((BM, NUM_EXPERTS), lambda i, j: (i, 0)),
        ],
        out_specs=pl.BlockSpec((BM, NUM_EXPERTS), lambda i, j: (i, 0)),
        out_shape=jax.ShapeDtypeStruct((N_TOKENS, NUM_EXPERTS), jnp.float32),
        scratch_shapes=[pltpu.VMEM((BM, 2 * NUM_EXPERTS), jnp.float32)],
        compiler_params=pltpu.CompilerParams(
            dimension_semantics=("parallel", "arbitrary")),
        interpret=_INTERPRET,
    )(x, w, b, noise)


def _sc_router_body(z_hbm, router_hbm, idx_hbm, z_v, out_v, idx_v):
    wid = lax.axis_index("s") * _NC + lax.axis_index("c")
    rbase = wid * _ROWS_PER_W
    pltpu.sync_copy(z_hbm.at[pl.ds(rbase, _ROWS_PER_W)], z_v)

    zeros16 = jnp.zeros((_L,), jnp.float32)

    def _zero_body(i, c):
        out_v[i // (NUM_EXPERTS // _L),
              pl.ds((i % (NUM_EXPERTS // _L)) * _L, _L)] = zeros16
        return c

    lax.fori_loop(0, _ZN // _L, _zero_body, 0, unroll=8)

    lanes = lax.iota(jnp.int32, _L)
    neg_inf = jnp.full((_L,), -jnp.inf, jnp.float32)
    zero_i = jnp.zeros((_L,), jnp.int32)

    def _group_body(g, c):
        row = g * _L + lanes
        vals, inds = [], []
        for k in range(TOP_K):
            def _scan(cstep, carry):
                bv, bi = carry
                for cc in range(8):
                    col = cstep * 8 + cc
                    v = plsc.load_gather(z_v, [row, zero_i + col])
                    gt = v > bv
                    bv = jnp.where(gt, v, bv)
                    bi = jnp.where(gt, col, bi)
                return bv, bi

            bv, bi = lax.fori_loop(0, NUM_EXPERTS // 8, _scan,
                                   (neg_inf, zero_i))
            plsc.store_scatter(z_v, [row, bi], neg_inf)
            vals.append(bv)
            inds.append(bi)
        mx = vals[0]
        es = [jnp.exp(v - mx) for v in vals]
        s = es[0]
        for e in es[1:]:
            s = s + e
        for k in range(TOP_K):
            plsc.store_scatter(out_v, [row, inds[k]], es[k] / s)
            plsc.store_scatter(idx_v, [row, zero_i + k], inds[k])
        return c

    lax.fori_loop(0, _GROUPS, _group_body, 0)

    pltpu.sync_copy(out_v, router_hbm.at[pl.ds(rbase, _ROWS_PER_W)])
    pltpu.sync_copy(idx_v, idx_hbm.at[pl.ds(rbase, _ROWS_PER_W)])


_sc_router = functools.partial(
    pl.kernel,
    out_type=[
        jax.ShapeDtypeStruct((N_TOKENS, NUM_EXPERTS), jnp.float32),
        jax.ShapeDtypeStruct((N_TOKENS, TOP_K), jnp.int32),
    ],
    mesh=plsc.VectorSubcoreMesh(core_axis_name="c", subcore_axis_name="s"),
    scratch_types=[
        pltpu.VMEM((_ROWS_PER_W, NUM_EXPERTS), jnp.float32),
        pltpu.VMEM((_ROWS_PER_W, NUM_EXPERTS), jnp.float32),
        pltpu.VMEM((_ROWS_PER_W, TOP_K), jnp.int32),
    ],
    compiler_params=pltpu.CompilerParams(needs_layout_passes=False),
)(_sc_router_body)


def kernel(x, W_gate, b_gate, W_noise, b_noise, rng_key):
    w = jnp.concatenate([W_gate, W_noise], axis=1)
    b = jnp.concatenate([b_gate, b_noise])[None, :]
    noise = jax.random.normal(rng_key, (N_TOKENS, NUM_EXPERTS),
                              dtype=jnp.float32)
    z = _noisy_logits(x, w, b, noise)
    router, idx = _sc_router(z)
    return (router, idx)
